# baseline (device time: 303272 ns/iter reference)
import jax
import jax.numpy as jnp
from jax import lax
from jax.experimental import pallas as pl
from jax.experimental.pallas import tpu as pltpu

N_DEV = 32


def kernel(t, W):
    m, k = t.shape
    _, n = W.shape
    c = m // N_DEV

    def body(t_ref, w_ref, out_ref, rs_send, rs_recv,
             rs_send_sems, rs_recv_sems, ag_send_sems, ag_recv_sems):
        d = lax.axis_index("i")
        left = (d - 1) % N_DEV
        right = (d + 1) % N_DEV

        def t_chunk(idx):
            return t_ref[pl.ds(idx * c, c), :]

        rs_send[0, :, :] = t_chunk(d)

        barrier_sem = pltpu.get_barrier_semaphore()
        for nbr in (left, right):
            pl.semaphore_signal(
                barrier_sem, inc=1,
                device_id=(nbr,), device_id_type=pl.DeviceIdType.MESH,
            )
        pl.semaphore_wait(barrier_sem, 2)

        acc = None
        for s in range(N_DEV - 1):
            rdma = pltpu.make_async_remote_copy(
                src_ref=rs_send.at[s],
                dst_ref=rs_recv.at[s],
                send_sem=rs_send_sems.at[s],
                recv_sem=rs_recv_sems.at[s],
                device_id=(right,),
                device_id_type=pl.DeviceIdType.MESH,
            )
            rdma.start()
            rdma.wait()
            idx = (d - s - 1) % N_DEV
            acc = rs_recv[s, :, :] + t_chunk(idx)
            if s < N_DEV - 2:
                rs_send[s + 1, :, :] = acc

        owned = (d + 1) % N_DEV
        out_ref[pl.ds(owned * c, c), :] = jnp.dot(
            acc, w_ref[:, :], preferred_element_type=jnp.float32
        )

        for h in range(N_DEV - 1):
            o = (d + 1 - h) % N_DEV
            rdma = pltpu.make_async_remote_copy(
                src_ref=out_ref.at[pl.ds(o * c, c)],
                dst_ref=out_ref.at[pl.ds(o * c, c)],
                send_sem=ag_send_sems.at[h],
                recv_sem=ag_recv_sems.at[h],
                device_id=(right,),
                device_id_type=pl.DeviceIdType.MESH,
            )
            rdma.start()
            rdma.wait()

    return pl.pallas_call(
        body,
        out_shape=jax.ShapeDtypeStruct((m, n), jnp.float32),
        in_specs=[
            pl.BlockSpec(memory_space=pltpu.VMEM),
            pl.BlockSpec(memory_space=pltpu.VMEM),
        ],
        out_specs=pl.BlockSpec(memory_space=pltpu.VMEM),
        scratch_shapes=[
            pltpu.VMEM((N_DEV - 1, c, k), jnp.float32),
            pltpu.VMEM((N_DEV - 1, c, k), jnp.float32),
            pltpu.SemaphoreType.DMA((N_DEV - 1,)),
            pltpu.SemaphoreType.DMA((N_DEV - 1,)),
            pltpu.SemaphoreType.DMA((N_DEV - 1,)),
            pltpu.SemaphoreType.DMA((N_DEV - 1,)),
        ],
        compiler_params=pltpu.CompilerParams(collective_id=0),
    )(t, W)


# device time: 257312 ns/iter; 1.1786x vs baseline; 1.1786x over previous
import jax
import jax.numpy as jnp
from jax import lax
from jax.experimental import pallas as pl
from jax.experimental.pallas import tpu as pltpu

N_DEV = 32
P = 2
NDIR = 2


def kernel(t, W):
    m, k = t.shape
    _, n = W.shape
    c = m // N_DEV
    hw = k // 2
    w = hw // P

    def col0(dir_, p):
        return dir_ * hw + p * w

    def body(t_ref, w_ref, out_ref, red_ref, send_buf, recv_buf,
             rs_ssem, rs_rsem, ag_ssem, ag_rsem):
        d = lax.axis_index("i")
        left = (d - 1) % N_DEV
        right = (d + 1) % N_DEV
        nbr_of = (right, left)

        def t_piece(idx, dir_, p):
            return t_ref[pl.ds(idx * c, c), pl.ds(col0(dir_, p), w)]

        def rs_chunk(dir_, s):
            return (d - s) % N_DEV if dir_ == 0 else (d + s + 2) % N_DEV

        def ag_origin(dir_, h):
            return (d + 1 - h) % N_DEV if dir_ == 0 else (d + h + 1) % N_DEV

        def rs_rdma(dir_, s, p):
            return pltpu.make_async_remote_copy(
                src_ref=send_buf.at[dir_, s, p],
                dst_ref=recv_buf.at[dir_, s, p],
                send_sem=rs_ssem.at[dir_, s, p],
                recv_sem=rs_rsem.at[dir_, s, p],
                device_id=(nbr_of[dir_],),
                device_id_type=pl.DeviceIdType.MESH,
            )

        def ag_rdma(dir_, h, p):
            o = ag_origin(dir_, h)
            sl = (pl.ds(o * c, c), pl.ds(col0(dir_, p), w))
            return pltpu.make_async_remote_copy(
                src_ref=out_ref.at[sl],
                dst_ref=out_ref.at[sl],
                send_sem=ag_ssem.at[dir_, h, p],
                recv_sem=ag_rsem.at[dir_, h, p],
                device_id=(nbr_of[dir_],),
                device_id_type=pl.DeviceIdType.MESH,
            )

        for dir_ in range(NDIR):
            for p in range(P):
                send_buf[dir_, 0, p] = t_piece(rs_chunk(dir_, 0), dir_, p)

        barrier_sem = pltpu.get_barrier_semaphore()
        for nbr in (left, right):
            pl.semaphore_signal(
                barrier_sem, inc=1,
                device_id=(nbr,), device_id_type=pl.DeviceIdType.MESH,
            )
        pl.semaphore_wait(barrier_sem, 2)

        for dir_ in range(NDIR):
            for p in range(P):
                rs_rdma(dir_, 0, p).start()
        for s in range(N_DEV - 1):
            for p in range(P):
                for dir_ in range(NDIR):
                    rdma = rs_rdma(dir_, s, p)
                    rdma.wait_recv()
                    idx = rs_chunk(dir_, s + 1)
                    acc = recv_buf[dir_, s, p] + t_piece(idx, dir_, p)
                    if s < N_DEV - 2:
                        send_buf[dir_, s + 1, p] = acc
                        rs_rdma(dir_, s + 1, p).start()
                    else:
                        red_ref[:, pl.ds(col0(dir_, p), w)] = acc
                    rdma.wait_send()

        owned = (d + 1) % N_DEV
        out_ref[pl.ds(owned * c, c), :] = jnp.dot(
            red_ref[:, :], w_ref[:, :], preferred_element_type=jnp.float32
        )

        for dir_ in range(NDIR):
            for p in range(P):
                ag_rdma(dir_, 0, p).start()
        for h in range(N_DEV - 1):
            for p in range(P):
                for dir_ in range(NDIR):
                    rdma = ag_rdma(dir_, h, p)
                    rdma.wait_recv()
                    if h < N_DEV - 2:
                        ag_rdma(dir_, h + 1, p).start()
                    rdma.wait_send()

    return pl.pallas_call(
        body,
        out_shape=jax.ShapeDtypeStruct((m, n), jnp.float32),
        in_specs=[
            pl.BlockSpec(memory_space=pltpu.VMEM),
            pl.BlockSpec(memory_space=pltpu.VMEM),
        ],
        out_specs=pl.BlockSpec(memory_space=pltpu.VMEM),
        scratch_shapes=[
            pltpu.VMEM((c, k), jnp.float32),
            pltpu.VMEM((NDIR, N_DEV - 1, P, c, w), jnp.float32),
            pltpu.VMEM((NDIR, N_DEV - 1, P, c, w), jnp.float32),
            pltpu.SemaphoreType.DMA((NDIR, N_DEV - 1, P)),
            pltpu.SemaphoreType.DMA((NDIR, N_DEV - 1, P)),
            pltpu.SemaphoreType.DMA((NDIR, N_DEV - 1, P)),
            pltpu.SemaphoreType.DMA((NDIR, N_DEV - 1, P)),
        ],
        compiler_params=pltpu.CompilerParams(collective_id=0),
    )(t, W)


# device time: 196536 ns/iter; 1.5431x vs baseline; 1.3092x over previous
import jax
import jax.numpy as jnp
from jax import lax
from jax.experimental import pallas as pl
from jax.experimental.pallas import tpu as pltpu

N_DEV = 32
N_ROW = 8
N_COL = 4
P = 2
NDIR = 2


def kernel(t, W):
    m, k = t.shape
    _, n = W.shape
    c1 = m // N_ROW
    c2 = c1 // N_COL
    hw = k // 2
    w = hw // P

    def col0(dir_, p):
        return dir_ * hw + p * w

    def body(t_ref, w_ref, out_ref, red1, red2,
             s1_send, s1_recv, s2_send, s2_recv,
             p1_ssem, p1_rsem, p2_ssem, p2_rsem,
             p3_ssem, p3_rsem, p4_ssem, p4_rsem):
        d = lax.axis_index("i")
        g = d // N_ROW
        r = d % N_ROW
        row_right = g * N_ROW + (r + 1) % N_ROW
        row_left = g * N_ROW + (r - 1) % N_ROW
        col_down = ((g + 1) % N_COL) * N_ROW + r
        col_up = ((g - 1) % N_COL) * N_ROW + r
        row_nbr = (row_right, row_left)
        col_nbr = (col_down, col_up)

        rho1 = (r + 1) % N_ROW
        gam1 = (g + 1) % N_COL

        def ring_rdma(src, dst, ssem, rsem, target):
            return pltpu.make_async_remote_copy(
                src_ref=src, dst_ref=dst, send_sem=ssem, recv_sem=rsem,
                device_id=(target,), device_id_type=pl.DeviceIdType.MESH,
            )

        def p1_chunk(dir_, s):
            return (r - s) % N_ROW if dir_ == 0 else (r + s + 2) % N_ROW

        def t_piece(rho, dir_, p):
            return t_ref[pl.ds(rho * c1, c1), pl.ds(col0(dir_, p), w)]

        def p1_rdma(dir_, s, p):
            return ring_rdma(
                s1_send.at[dir_, s, p], s1_recv.at[dir_, s, p],
                p1_ssem.at[dir_, s, p], p1_rsem.at[dir_, s, p],
                row_nbr[dir_])

        def p2_chunk(dir_, s):
            return (g - s) % N_COL if dir_ == 0 else (g + s + 2) % N_COL

        def red1_piece(gam, dir_, p):
            return red1[pl.ds(gam * c2, c2), pl.ds(col0(dir_, p), w)]

        def p2_rdma(dir_, s, p):
            return ring_rdma(
                s2_send.at[dir_, s, p], s2_recv.at[dir_, s, p],
                p2_ssem.at[dir_, s, p], p2_rsem.at[dir_, s, p],
                col_nbr[dir_])

        def p3_rdma(dir_, h, p):
            gam = ((g + 1 - h) if dir_ == 0 else (g + h + 1)) % N_COL
            sl = (pl.ds(rho1 * c1 + gam * c2, c2), pl.ds(col0(dir_, p), w))
            return ring_rdma(
                out_ref.at[sl], out_ref.at[sl],
                p3_ssem.at[dir_, h, p], p3_rsem.at[dir_, h, p],
                col_nbr[dir_])

        def p4_rdma(dir_, h, p):
            rho = ((r + 1 - h) if dir_ == 0 else (r + h + 1)) % N_ROW
            sl = (pl.ds(rho * c1, c1), pl.ds(col0(dir_, p), w))
            return ring_rdma(
                out_ref.at[sl], out_ref.at[sl],
                p4_ssem.at[dir_, h, p], p4_rsem.at[dir_, h, p],
                row_nbr[dir_])

        for dir_ in range(NDIR):
            for p in range(P):
                s1_send[dir_, 0, p] = t_piece(p1_chunk(dir_, 0), dir_, p)

        barrier_sem = pltpu.get_barrier_semaphore()
        for nbr in (row_left, row_right, col_up, col_down):
            pl.semaphore_signal(
                barrier_sem, inc=1,
                device_id=(nbr,), device_id_type=pl.DeviceIdType.MESH,
            )
        pl.semaphore_wait(barrier_sem, 4)

        for dir_ in range(NDIR):
            for p in range(P):
                p1_rdma(dir_, 0, p).start()
        for s in range(N_ROW - 1):
            for p in range(P):
                for dir_ in range(NDIR):
                    rdma = p1_rdma(dir_, s, p)
                    rdma.wait_recv()
                    acc = s1_recv[dir_, s, p] + t_piece(
                        p1_chunk(dir_, s + 1), dir_, p)
                    if s < N_ROW - 2:
                        s1_send[dir_, s + 1, p] = acc
                        p1_rdma(dir_, s + 1, p).start()
                    else:
                        red1[:, pl.ds(col0(dir_, p), w)] = acc
                    rdma.wait_send()

        for dir_ in range(NDIR):
            for p in range(P):
                s2_send[dir_, 0, p] = red1_piece(p2_chunk(dir_, 0), dir_, p)
                p2_rdma(dir_, 0, p).start()
        for s in range(N_COL - 1):
            for p in range(P):
                for dir_ in range(NDIR):
                    rdma = p2_rdma(dir_, s, p)
                    rdma.wait_recv()
                    acc = s2_recv[dir_, s, p] + red1_piece(
                        p2_chunk(dir_, s + 1), dir_, p)
                    if s < N_COL - 2:
                        s2_send[dir_, s + 1, p] = acc
                        p2_rdma(dir_, s + 1, p).start()
                    else:
                        red2[:, pl.ds(col0(dir_, p), w)] = acc
                    rdma.wait_send()

        out_ref[pl.ds(rho1 * c1 + gam1 * c2, c2), :] = jnp.dot(
            red2[:, :], w_ref[:, :], preferred_element_type=jnp.float32
        )

        for dir_ in range(NDIR):
            for p in range(P):
                p3_rdma(dir_, 0, p).start()
        for h in range(N_COL - 1):
            for p in range(P):
                for dir_ in range(NDIR):
                    rdma = p3_rdma(dir_, h, p)
                    rdma.wait_recv()
                    if h < N_COL - 2:
                        p3_rdma(dir_, h + 1, p).start()
                    rdma.wait_send()

        for dir_ in range(NDIR):
            for p in range(P):
                p4_rdma(dir_, 0, p).start()
        for h in range(N_ROW - 1):
            for p in range(P):
                for dir_ in range(NDIR):
                    rdma = p4_rdma(dir_, h, p)
                    rdma.wait_recv()
                    if h < N_ROW - 2:
                        p4_rdma(dir_, h + 1, p).start()
                    rdma.wait_send()

    return pl.pallas_call(
        body,
        out_shape=jax.ShapeDtypeStruct((m, n), jnp.float32),
        in_specs=[
            pl.BlockSpec(memory_space=pltpu.VMEM),
            pl.BlockSpec(memory_space=pltpu.VMEM),
        ],
        out_specs=pl.BlockSpec(memory_space=pltpu.VMEM),
        scratch_shapes=[
            pltpu.VMEM((c1, k), jnp.float32),
            pltpu.VMEM((c2, k), jnp.float32),
            pltpu.VMEM((NDIR, N_ROW - 1, P, c1, w), jnp.float32),
            pltpu.VMEM((NDIR, N_ROW - 1, P, c1, w), jnp.float32),
            pltpu.VMEM((NDIR, N_COL - 1, P, c2, w), jnp.float32),
            pltpu.VMEM((NDIR, N_COL - 1, P, c2, w), jnp.float32),
            pltpu.SemaphoreType.DMA((NDIR, N_ROW - 1, P)),
            pltpu.SemaphoreType.DMA((NDIR, N_ROW - 1, P)),
            pltpu.SemaphoreType.DMA((NDIR, N_COL - 1, P)),
            pltpu.SemaphoreType.DMA((NDIR, N_COL - 1, P)),
            pltpu.SemaphoreType.DMA((NDIR, N_COL - 1, P)),
            pltpu.SemaphoreType.DMA((NDIR, N_COL - 1, P)),
            pltpu.SemaphoreType.DMA((NDIR, N_ROW - 1, P)),
            pltpu.SemaphoreType.DMA((NDIR, N_ROW - 1, P)),
        ],
        compiler_params=pltpu.CompilerParams(collective_id=0),
    )(t, W)
